# baseline (speedup 1.0000x reference)
import jax
import jax.numpy as jnp
from jax.experimental import pallas as pl

N_PAD_SLOT = 511
P = 512


def _dense_body(xs_ref, xd_ref, dcol_ref, drow_ref, w1_ref, a1s_ref, a1d_ref,
                w2_ref, a2s_ref, a2d_ref, hc2_ref, qs_ref, qd_ref):
    xs = xs_ref[...]
    xd = xd_ref[...]
    W1 = w1_ref[...]
    W2 = w2_ref[...]
    f32 = jnp.float32
    dot = lambda a, b: jax.lax.dot(a, b, preferred_element_type=f32,
                                   precision=jax.lax.Precision.HIGHEST)
    b1s = dot(W1, a1s_ref[...])
    b1d = dot(W1, a1d_ref[...])
    e1 = dot(xs, b1s) + dot(xd, b1d)
    e1 = jnp.where(e1 >= 0, e1, 0.2 * e1)
    valid = jax.lax.broadcasted_iota(jnp.int32, (P, 1), 0) < 500
    w1 = jnp.where(valid, jnp.exp(e1), 0.0)
    hs1 = dot(xs, W1)
    M = (dcol_ref[...] == drow_ref[...]).astype(f32)
    dvec = dot(M, w1)
    r = w1 / (dvec + 1e-30)
    Hc = dot(M, r * hs1)
    Hc2 = dot(Hc, W2)
    hc2_ref[...] = Hc2
    qs_ref[...] = dot(Hc2, a2s_ref[...])
    qd_ref[...] = dot(Hc2, a2d_ref[...])


def _dense_stage(xs, xd, d1, W1, a_src1, a_dst1, W2, a_src2, a_dst2):
    f32 = jnp.float32
    out_shapes = (
        jax.ShapeDtypeStruct((P, 128), f32),
        jax.ShapeDtypeStruct((P, 1), f32),
        jax.ShapeDtypeStruct((P, 1), f32),
    )
    return pl.pallas_call(_dense_body, out_shape=out_shapes)(
        xs, xd, d1.reshape(P, 1), d1.reshape(1, P),
        W1, a_src1.reshape(128, 1), a_dst1.reshape(128, 1),
        W2, a_src2.reshape(128, 1), a_dst2.reshape(128, 1))


def kernel(x, edge_index, W1, a_src1, a_dst1, W2, a_src2, a_dst2):
    N = x.shape[0]
    src1 = edge_index[0, :500]
    dst1 = edge_index[1, :500]
    src2 = edge_index[0, 500:]
    dst2 = edge_index[1, 500:]

    pad_i = jnp.zeros((P - 500,), jnp.int32)
    d1 = jnp.concatenate([dst1, jnp.full((P - 500,), N, jnp.int32)])
    xs = x[jnp.concatenate([src1, pad_i])]
    xd = x[jnp.concatenate([dst1, pad_i])]

    Hc2, q_src, q_dst = _dense_stage(xs, xd, d1, W1, a_src1, a_dst1,
                                     W2, a_src2, a_dst2)
    q_src = q_src[:, 0]
    q_dst = q_dst[:, 0]

    ptab = jnp.full((N,), N_PAD_SLOT, jnp.int32).at[dst1].set(
        jnp.arange(500, dtype=jnp.int32))

    slot_s = ptab[src2]
    slot_d = ptab[dst2]
    e2 = q_src[slot_s] + q_dst[slot_d]
    e2 = jnp.where(e2 >= 0, e2, 0.2 * e2)
    xv = jnp.exp(e2)
    denom = jax.ops.segment_sum(xv, dst2, num_segments=N)
    alpha = xv / (denom[dst2] + 1e-16)
    out = jax.ops.segment_sum(alpha[:, None] * Hc2[slot_s], dst2,
                              num_segments=N)
    return out


# speedup vs baseline: 103.0327x; 103.0327x over previous
import functools

import jax
import jax.numpy as jnp
from jax import lax
from jax.experimental import pallas as pl
from jax.experimental.pallas import tpu as pltpu
from jax.experimental.pallas import tpu_sc as plsc

N = 10000
D = 128
NE1 = 500
NE2 = 320000 - NE1
P = 512
PAD_SLOT = 511

NC, NS, L = 2, 16, 16
NW = NC * NS
CHUNK = 10112
NWIN = CHUNK // 128
EPAD = NW * CHUNK
NPT = 10240
CCAP = 10240
WROWS = 1024

_mesh = plsc.VectorSubcoreMesh(core_axis_name="c", subcore_axis_name="s")
_NLP = pltpu.CompilerParams(needs_layout_passes=False)


def _wid():
    return lax.axis_index("s") * NC + lax.axis_index("c")


@functools.partial(
    pl.kernel,
    out_type=(
        jax.ShapeDtypeStruct((P, D), jnp.float32),
        jax.ShapeDtypeStruct((P, D), jnp.float32),
        jax.ShapeDtypeStruct((NPT,), jnp.int32),
    ),
    mesh=_mesh,
    compiler_params=_NLP,
    scratch_types=[
        pltpu.VMEM((L,), jnp.int32),
        pltpu.VMEM((L, D), jnp.float32),
        pltpu.VMEM((P,), jnp.int32),
        pltpu.VMEM((NPT,), jnp.int32),
        pltpu.SemaphoreType.DMA,
    ],
)
def _sc_prep(x_hbm, s1_hbm, d1_hbm, xs_hbm, xd_hbm, ptab_hbm,
             idxb, rowb, dstb, ptb, sem):
    wid = _wid()
    base = wid * L
    pltpu.sync_copy(s1_hbm.at[pl.ds(base, L)], idxb)
    pltpu.async_copy(x_hbm.at[idxb], rowb, sem).wait()
    pltpu.sync_copy(rowb, xs_hbm.at[pl.ds(base, L)])
    pltpu.sync_copy(d1_hbm.at[pl.ds(base, L)], idxb)
    pltpu.async_copy(x_hbm.at[idxb], rowb, sem).wait()
    pltpu.sync_copy(rowb, xd_hbm.at[pl.ds(base, L)])

    @pl.when(wid == 0)
    def _():
        pltpu.sync_copy(d1_hbm, dstb)
        fill = jnp.full((L,), PAD_SLOT, jnp.int32)

        def init(i, _):
            ptb[pl.ds(i * L, L)] = fill
            return 0
        lax.fori_loop(0, NPT // L, init, 0)

        lanes = lax.iota(jnp.int32, L)

        def scat(b, _):
            d = dstb[pl.ds(b * L, L)]
            j = jnp.full((L,), b * L, jnp.int32) + lanes
            for l in range(L):
                plsc.store_scatter(ptb, [d], j, mask=lanes == l)
            return 0
        lax.fori_loop(0, P // L, scat, 0)
        pltpu.sync_copy(ptb, ptab_hbm)


def _dense_body(xs_ref, xd_ref, dcol_ref, drow_ref, w1_ref, a1s_ref, a1d_ref,
                w2_ref, a2s_ref, a2d_ref, hc2_ref, qs_ref, qd_ref):
    xs = xs_ref[...]
    xd = xd_ref[...]
    W1 = w1_ref[...]
    W2 = w2_ref[...]
    f32 = jnp.float32
    dot = lambda a, b: jax.lax.dot(a, b, preferred_element_type=f32,
                                   precision=jax.lax.Precision.HIGHEST)
    b1s = dot(W1, a1s_ref[...])
    b1d = dot(W1, a1d_ref[...])
    e1 = dot(xs, b1s) + dot(xd, b1d)
    e1 = jnp.where(e1 >= 0, e1, 0.2 * e1)
    valid = jax.lax.broadcasted_iota(jnp.int32, (P, 1), 0) < NE1
    w1 = jnp.where(valid, jnp.exp(e1), 0.0)
    hs1 = dot(xs, W1)
    M = (dcol_ref[...] == drow_ref[...]).astype(f32)
    dvec = dot(M, w1)
    r = w1 / (dvec + 1e-30)
    Hc = dot(M, r * hs1)
    Hc2 = dot(Hc, W2)
    hc2_ref[...] = Hc2
    qs_ref[...] = dot(Hc2, a2s_ref[...])
    qd_ref[...] = dot(Hc2, a2d_ref[...])


def _dense_stage(xs, xd, d1, W1, a_src1, a_dst1, W2, a_src2, a_dst2):
    f32 = jnp.float32
    out_shapes = (
        jax.ShapeDtypeStruct((P, D), f32),
        jax.ShapeDtypeStruct((P, 1), f32),
        jax.ShapeDtypeStruct((P, 1), f32),
    )
    return pl.pallas_call(_dense_body, out_shape=out_shapes)(
        xs, xd, d1.reshape(P, 1), d1.reshape(1, P),
        W1, a_src1.reshape(D, 1), a_dst1.reshape(D, 1),
        W2, a_src2.reshape(D, 1), a_dst2.reshape(D, 1))


@functools.partial(
    pl.kernel,
    out_type=(
        jax.ShapeDtypeStruct((NC, NPT), jnp.float32),
        jax.ShapeDtypeStruct((NW, CCAP), jnp.int32),
        jax.ShapeDtypeStruct((NW, CCAP), jnp.int32),
        jax.ShapeDtypeStruct((NW, CCAP), jnp.float32),
        jax.ShapeDtypeStruct((NW, L), jnp.int32),
    ),
    mesh=_mesh,
    compiler_params=_NLP,
    scratch_types=[
        pltpu.VMEM((CHUNK,), jnp.int32),
        pltpu.VMEM((CHUNK,), jnp.int32),
        pltpu.VMEM((NPT,), jnp.int32),
        pltpu.VMEM((P,), jnp.float32),
        pltpu.VMEM((P,), jnp.float32),
        pltpu.VMEM((CCAP,), jnp.int32),
        pltpu.VMEM((CCAP,), jnp.int32),
        pltpu.VMEM((CCAP,), jnp.float32),
        pltpu.VMEM((128,), jnp.int32),
        pltpu.VMEM((128,), jnp.float32),
        pltpu.VMEM((L,), jnp.int32),
        pltpu.VMEM((1024,), jnp.float32),
        pltpu.VMEM_SHARED((NPT,), jnp.float32),
    ],
)
def _sc_edges(src_hbm, dst_hbm, ptab_hbm, qs_hbm, qd_hbm,
              denp_hbm, cdst_hbm, cslot_hbm, cxv_hbm, cnt_hbm,
              srcb, dstb, ptb, qsb, qdb, cdst, cslot, cxv,
              widx, wval, cntb, zb, den_sh):
    c = lax.axis_index("c")
    s = lax.axis_index("s")
    wid = s * NC + c
    eb = wid * CHUNK
    lanes = lax.iota(jnp.int32, L)
    zero16 = jnp.zeros((L,), jnp.float32)

    @pl.when(s == 0)
    def _():
        def zloop(i, _):
            zb[pl.ds(i * L, L)] = zero16
            return 0
        lax.fori_loop(0, 1024 // L, zloop, 0)

        def zcopy(i, _):
            pltpu.sync_copy(zb, den_sh.at[pl.ds(i * 1024, 1024)])
            return 0
        lax.fori_loop(0, NPT // 1024, zcopy, 0)
    plsc.subcore_barrier()

    pltpu.sync_copy(src_hbm.at[pl.ds(eb, CHUNK)], srcb)
    pltpu.sync_copy(dst_hbm.at[pl.ds(eb, CHUNK)], dstb)
    pltpu.sync_copy(ptab_hbm, ptb)
    pltpu.sync_copy(qs_hbm, qsb)
    pltpu.sync_copy(qd_hbm, qdb)

    def window(w, cnt_vec):
        for k in range(8):
            off = w * 128 + k * L
            sv = srcb[pl.ds(off, L)]
            dv = dstb[pl.ds(off, L)]
            ss = plsc.load_gather(ptb, [sv])
            sd = plsc.load_gather(ptb, [dv])
            e = plsc.load_gather(qsb, [ss]) + plsc.load_gather(qdb, [sd])
            e = jnp.where(e >= 0, e, 0.2 * e)
            xv = jnp.exp(e)
            gid = jnp.full((L,), eb + off, jnp.int32) + lanes
            validm = gid < NE2
            xv = jnp.where(validm, xv, 0.0)
            widx[pl.ds(k * L, L)] = dv
            wval[pl.ds(k * L, L)] = xv
            hot = validm & (ss < NE1)
            pos = cnt_vec + plsc.cumsum(hot.astype(jnp.int32)) - 1
            plsc.store_scatter(cdst, [pos], dv, mask=hot)
            plsc.store_scatter(cslot, [pos], ss, mask=hot)
            plsc.store_scatter(cxv, [pos], xv, mask=hot)
            cnt_vec = cnt_vec + plsc.all_reduce_population_count(hot)
        pltpu.sync_copy(wval, den_sh.at[widx], add=True)
        return cnt_vec

    cnt_vec = lax.fori_loop(0, NWIN, window, jnp.zeros((L,), jnp.int32))

    pad_pos = cnt_vec + lanes
    plsc.store_scatter(cdst, [pad_pos], jnp.zeros((L,), jnp.int32))
    plsc.store_scatter(cslot, [pad_pos], jnp.full((L,), PAD_SLOT, jnp.int32))
    plsc.store_scatter(cxv, [pad_pos], zero16)
    cntb[...] = cnt_vec

    pltpu.sync_copy(cdst, cdst_hbm.at[wid])
    pltpu.sync_copy(cslot, cslot_hbm.at[wid])
    pltpu.sync_copy(cxv, cxv_hbm.at[wid])
    pltpu.sync_copy(cntb, cnt_hbm.at[wid])

    plsc.subcore_barrier()

    @pl.when(s == 0)
    def _():
        pltpu.sync_copy(den_sh, denp_hbm.at[c])


@functools.partial(
    pl.kernel,
    out_type=jax.ShapeDtypeStruct((NC, N, D), jnp.float32),
    mesh=_mesh,
    compiler_params=_NLP,
    scratch_types=[
        pltpu.VMEM((NPT,), jnp.float32),
        pltpu.VMEM((2048,), jnp.float32),
        pltpu.VMEM((WROWS,), jnp.int32),
        pltpu.VMEM((WROWS,), jnp.int32),
        pltpu.VMEM((WROWS,), jnp.float32),
        pltpu.VMEM((L, D), jnp.float32),
        pltpu.VMEM((L,), jnp.int32),
        pltpu.VMEM((L,), jnp.int32),
        pltpu.VMEM((L,), jnp.int32),
        pltpu.SemaphoreType.DMA,
        pltpu.VMEM_SHARED((N, D), jnp.float32),
    ],
)
def _sc_rows(hc2_hbm, denp_hbm, cdst_hbm, cslot_hbm, cxv_hbm, cnt_hbm,
             outp_hbm, db, dtmp, wdst, wslot, wxv,
             rowb, ridx, sidx, cntb, sem, out_sh):
    c = lax.axis_index("c")
    s = lax.axis_index("s")
    wid = s * NC + c
    lanes = lax.iota(jnp.int32, L)

    pltpu.sync_copy(cnt_hbm.at[wid], cntb)
    cnt = jnp.max(cntb[...])

    pltpu.sync_copy(denp_hbm.at[0], db)

    def dsum(i, _):
        pltpu.sync_copy(denp_hbm.at[1].at[pl.ds(i * 2048, 2048)], dtmp)

        def dadd(k, _):
            o = i * 2048 + k * L
            db[pl.ds(o, L)] = db[pl.ds(o, L)] + dtmp[pl.ds(k * L, L)]
            return 0
        lax.fori_loop(0, 2048 // L, dadd, 0)
        return 0
    lax.fori_loop(0, NPT // 2048, dsum, 0)

    zero16f = jnp.zeros((L,), jnp.float32)
    for i in range(L):
        for j in range(D // L):
            rowb[i, pl.ds(j * L, L)] = zero16f
    rows_per = N // NS

    def zc(i, _):
        pltpu.sync_copy(rowb, out_sh.at[pl.ds(s * rows_per + i * L, L)])
        return 0
    lax.fori_loop(0, rows_per // L, zc, 0)
    pltpu.sync_copy(rowb.at[pl.ds(0, 1)],
                    out_sh.at[pl.ds(s * rows_per + rows_per - 1, 1)])
    plsc.subcore_barrier()

    nwin = (cnt + WROWS - 1) // WROWS

    def window(w, _):
        pltpu.sync_copy(cdst_hbm.at[wid].at[pl.ds(w * WROWS, WROWS)], wdst)
        pltpu.sync_copy(cslot_hbm.at[wid].at[pl.ds(w * WROWS, WROWS)], wslot)
        pltpu.sync_copy(cxv_hbm.at[wid].at[pl.ds(w * WROWS, WROWS)], wxv)
        nb = (jnp.minimum(cnt - w * WROWS, WROWS) + L - 1) // L

        def body(b, _):
            off = b * L
            dv = wdst[pl.ds(off, L)]
            sv = wslot[pl.ds(off, L)]
            xv = wxv[pl.ds(off, L)]
            den = plsc.load_gather(db, [dv])
            al = xv / (den + 1e-16)
            ridx[...] = dv
            sidx[...] = sv
            pltpu.async_copy(hc2_hbm.at[sidx], rowb, sem).wait()
            for i in range(L):
                af = al[i]
                for j in range(D // L):
                    sl = pl.ds(j * L, L)
                    rowb[i, sl] = rowb[i, sl] * af
            pltpu.sync_copy(rowb, out_sh.at[ridx], add=True)
            return 0
        lax.fori_loop(0, nb, body, 0)
        return 0
    lax.fori_loop(0, nwin, window, 0)

    plsc.subcore_barrier()

    @pl.when(s == 0)
    def _():
        pltpu.sync_copy(out_sh, outp_hbm.at[c])


def _combine_body(p_ref, o_ref):
    o_ref[...] = p_ref[0] + p_ref[1]


def _combine(outp):
    blk = 2000
    return pl.pallas_call(
        _combine_body,
        grid=(N // blk,),
        in_specs=[pl.BlockSpec((NC, blk, D), lambda i: (0, i, 0))],
        out_specs=pl.BlockSpec((blk, D), lambda i: (i, 0)),
        out_shape=jax.ShapeDtypeStruct((N, D), jnp.float32),
    )(outp)


def kernel(x, edge_index, W1, a_src1, a_dst1, W2, a_src2, a_dst2):
    src1 = edge_index[0, :NE1]
    dst1 = edge_index[1, :NE1]
    src2 = edge_index[0, NE1:]
    dst2 = edge_index[1, NE1:]

    pad_i = jnp.zeros((P - NE1,), jnp.int32)
    s1 = jnp.concatenate([src1, pad_i])
    d1 = jnp.concatenate([dst1, jnp.full((P - NE1,), N, jnp.int32)])

    xs, xd, ptab = _sc_prep(x, s1, d1)

    Hc2, q_src, q_dst = _dense_stage(xs, xd, d1, W1, a_src1, a_dst1,
                                     W2, a_src2, a_dst2)

    epad = jnp.zeros((EPAD - NE2,), jnp.int32)
    src2p = jnp.concatenate([src2, epad])
    dst2p = jnp.concatenate([dst2, epad])

    denp, cdst, cslot, cxv, cnts = _sc_edges(
        src2p, dst2p, ptab, q_src.reshape(P), q_dst.reshape(P))

    outp = _sc_rows(Hc2, denp, cdst, cslot, cxv, cnts)
    return _combine(outp)


# speedup vs baseline: 106.9719x; 1.0382x over previous
import functools

import jax
import jax.numpy as jnp
from jax import lax
from jax.experimental import pallas as pl
from jax.experimental.pallas import tpu as pltpu
from jax.experimental.pallas import tpu_sc as plsc

N = 10000
D = 128
NE1 = 500
NE2 = 320000 - NE1
P = 512
PAD_SLOT = 511

NC, NS, L = 2, 16, 16
NW = NC * NS
CHUNK = 10112
WDEN = 1264
EPAD = NW * CHUNK
NPT = 10240
CCAP = CHUNK + L

_mesh = plsc.VectorSubcoreMesh(core_axis_name="c", subcore_axis_name="s")
_NLP = pltpu.CompilerParams(needs_layout_passes=False)


def _wid():
    return lax.axis_index("s") * NC + lax.axis_index("c")


@functools.partial(
    pl.kernel,
    out_type=(
        jax.ShapeDtypeStruct((P, D), jnp.float32),
        jax.ShapeDtypeStruct((P, D), jnp.float32),
        jax.ShapeDtypeStruct((NPT,), jnp.int32),
    ),
    mesh=_mesh,
    compiler_params=_NLP,
    scratch_types=[
        pltpu.VMEM((L,), jnp.int32),
        pltpu.VMEM((L, D), jnp.float32),
        pltpu.VMEM((P,), jnp.int32),
        pltpu.VMEM((NPT,), jnp.int32),
        pltpu.SemaphoreType.DMA,
    ],
)
def _sc_prep(x_hbm, s1_hbm, d1_hbm, xs_hbm, xd_hbm, ptab_hbm,
             idxb, rowb, dstb, ptb, sem):
    wid = _wid()
    base = wid * L
    pltpu.sync_copy(s1_hbm.at[pl.ds(base, L)], idxb)
    pltpu.async_copy(x_hbm.at[idxb], rowb, sem).wait()
    pltpu.sync_copy(rowb, xs_hbm.at[pl.ds(base, L)])
    pltpu.sync_copy(d1_hbm.at[pl.ds(base, L)], idxb)
    pltpu.async_copy(x_hbm.at[idxb], rowb, sem).wait()
    pltpu.sync_copy(rowb, xd_hbm.at[pl.ds(base, L)])

    @pl.when(wid == 0)
    def _():
        pltpu.sync_copy(d1_hbm, dstb)
        fill = jnp.full((L,), PAD_SLOT, jnp.int32)

        def init(i, _):
            ptb[pl.ds(i * L, L)] = fill
            return 0
        lax.fori_loop(0, NPT // L, init, 0)

        lanes = lax.iota(jnp.int32, L)

        def scat(b, _):
            d = dstb[pl.ds(b * L, L)]
            j = jnp.full((L,), b * L, jnp.int32) + lanes
            for l in range(L):
                plsc.store_scatter(ptb, [d], j, mask=lanes == l)
            return 0
        lax.fori_loop(0, P // L, scat, 0)
        pltpu.sync_copy(ptb, ptab_hbm)


def _dense_body(xs_ref, xd_ref, dcol_ref, drow_ref, w1_ref, a1s_ref, a1d_ref,
                w2_ref, a2s_ref, a2d_ref, hc2_ref, qs_ref, qd_ref):
    xs = xs_ref[...]
    xd = xd_ref[...]
    W1 = w1_ref[...]
    W2 = w2_ref[...]
    f32 = jnp.float32
    dot = lambda a, b: jax.lax.dot(a, b, preferred_element_type=f32,
                                   precision=jax.lax.Precision.HIGHEST)
    b1s = dot(W1, a1s_ref[...])
    b1d = dot(W1, a1d_ref[...])
    e1 = dot(xs, b1s) + dot(xd, b1d)
    e1 = jnp.where(e1 >= 0, e1, 0.2 * e1)
    valid = jax.lax.broadcasted_iota(jnp.int32, (P, 1), 0) < NE1
    w1 = jnp.where(valid, jnp.exp(e1), 0.0)
    hs1 = dot(xs, W1)
    M = (dcol_ref[...] == drow_ref[...]).astype(f32)
    dvec = dot(M, w1)
    r = w1 / (dvec + 1e-30)
    Hc = dot(M, r * hs1)
    Hc2 = dot(Hc, W2)
    hc2_ref[...] = Hc2
    qs_ref[...] = dot(Hc2, a2s_ref[...])
    qd_ref[...] = dot(Hc2, a2d_ref[...])


def _dense_stage(xs, xd, d1, W1, a_src1, a_dst1, W2, a_src2, a_dst2):
    f32 = jnp.float32
    out_shapes = (
        jax.ShapeDtypeStruct((P, D), f32),
        jax.ShapeDtypeStruct((P, 1), f32),
        jax.ShapeDtypeStruct((P, 1), f32),
    )
    return pl.pallas_call(_dense_body, out_shape=out_shapes)(
        xs, xd, d1.reshape(P, 1), d1.reshape(1, P),
        W1, a_src1.reshape(D, 1), a_dst1.reshape(D, 1),
        W2, a_src2.reshape(D, 1), a_dst2.reshape(D, 1))


@functools.partial(
    pl.kernel,
    out_type=(
        jax.ShapeDtypeStruct((NC, NPT), jnp.float32),
        jax.ShapeDtypeStruct((NC, N, D), jnp.float32),
    ),
    mesh=_mesh,
    compiler_params=_NLP,
    scratch_types=[
        pltpu.VMEM((WDEN,), jnp.int32),
        pltpu.VMEM((WDEN,), jnp.int32),
        pltpu.VMEM((NPT,), jnp.int32),
        pltpu.VMEM((P,), jnp.float32),
        pltpu.VMEM((P,), jnp.float32),
        pltpu.VMEM((CCAP,), jnp.int32),
        pltpu.VMEM((CCAP,), jnp.int32),
        pltpu.VMEM((CCAP,), jnp.float32),
        pltpu.VMEM((WDEN,), jnp.int32),
        pltpu.VMEM((WDEN,), jnp.float32),
        pltpu.VMEM((L, D), jnp.float32),
        pltpu.VMEM((L,), jnp.int32),
        pltpu.VMEM((L,), jnp.int32),
        pltpu.SemaphoreType.DMA,
        pltpu.VMEM_SHARED((NPT,), jnp.float32),
        pltpu.VMEM_SHARED((N, D), jnp.float32),
    ],
)
def _sc_main(src_hbm, dst_hbm, ptab_hbm, qs_hbm, qd_hbm, hc2_hbm,
             denp_hbm, outp_hbm,
             srcb, dstb, ptb, qsb, qdb, cdst, cslot, cxv,
             widx, wval, rowb, ridx, sidx, sem, den_sh, out_sh):
    c = lax.axis_index("c")
    s = lax.axis_index("s")
    wid = s * NC + c
    eb = wid * CHUNK
    lanes = lax.iota(jnp.int32, L)
    zero16 = jnp.zeros((L,), jnp.float32)

    for i in range(L):
        for j in range(D // L):
            rowb[i, pl.ds(j * L, L)] = zero16
    rows_per = N // NS

    def zrow(k, _):
        pltpu.sync_copy(rowb, out_sh.at[pl.ds(s * rows_per + k * L, L)])
        return 0
    lax.fori_loop(0, rows_per // L, zrow, 0)
    rem = rows_per % L
    pltpu.sync_copy(
        rowb.at[pl.ds(0, rem)],
        out_sh.at[pl.ds(s * rows_per + (rows_per // L) * L, rem)])

    @pl.when(s == 0)
    def _():
        def zl(i, _):
            wval[pl.ds(i * L, L)] = zero16
            return 0
        lax.fori_loop(0, 1024 // L, zl, 0)

        def zc(i, _):
            pltpu.sync_copy(wval.at[pl.ds(0, 1024)],
                            den_sh.at[pl.ds(i * 1024, 1024)])
            return 0
        lax.fori_loop(0, NPT // 1024, zc, 0)

    pltpu.sync_copy(ptab_hbm, ptb)
    pltpu.sync_copy(qs_hbm, qsb)
    pltpu.sync_copy(qd_hbm, qdb)
    plsc.subcore_barrier()

    def window(w, cnt_vec):
        pltpu.sync_copy(src_hbm.at[pl.ds(eb + w * WDEN, WDEN)], srcb)
        pltpu.sync_copy(dst_hbm.at[pl.ds(eb + w * WDEN, WDEN)], dstb)

        def batch(k, cv):
            off = k * L
            sv = srcb[pl.ds(off, L)]
            dv = dstb[pl.ds(off, L)]
            ss = plsc.load_gather(ptb, [sv])
            sd = plsc.load_gather(ptb, [dv])
            e = plsc.load_gather(qsb, [ss]) + plsc.load_gather(qdb, [sd])
            e = jnp.where(e >= 0, e, 0.2 * e)
            xv = jnp.exp(e)
            gid = jnp.full((L,), eb + w * WDEN, jnp.int32) + off + lanes
            validm = gid < NE2
            xv = jnp.where(validm, xv, 0.0)
            widx[pl.ds(k * L, L)] = dv
            wval[pl.ds(k * L, L)] = xv
            hot = validm & (ss < NE1)
            pos = cv + plsc.cumsum(hot.astype(jnp.int32)) - 1
            plsc.store_scatter(cdst, [pos], dv, mask=hot)
            plsc.store_scatter(cslot, [pos], ss, mask=hot)
            plsc.store_scatter(cxv, [pos], xv, mask=hot)
            return cv + plsc.all_reduce_population_count(hot)
        cnt_vec = lax.fori_loop(0, WDEN // L, batch, cnt_vec)
        pltpu.sync_copy(wval, den_sh.at[widx], add=True)
        return cnt_vec

    cnt_vec = lax.fori_loop(0, CHUNK // WDEN, window,
                            jnp.zeros((L,), jnp.int32))

    pad_pos = cnt_vec + lanes
    plsc.store_scatter(cdst, [pad_pos], jnp.zeros((L,), jnp.int32))
    plsc.store_scatter(cslot, [pad_pos], jnp.full((L,), PAD_SLOT, jnp.int32))
    plsc.store_scatter(cxv, [pad_pos], zero16)

    plsc.subcore_barrier()

    @pl.when(s == 0)
    def _():
        pltpu.sync_copy(den_sh, denp_hbm.at[c])

    cnt = jnp.max(cnt_vec)
    nb = (cnt + L - 1) // L

    def body(b, _):
        off = b * L
        dv = cdst[pl.ds(off, L)]
        sv = cslot[pl.ds(off, L)]
        xv = cxv[pl.ds(off, L)]
        ridx[...] = dv
        sidx[...] = sv
        pltpu.async_copy(hc2_hbm.at[sidx], rowb, sem).wait()
        for i in range(L):
            af = xv[i]
            for j in range(D // L):
                sl = pl.ds(j * L, L)
                rowb[i, sl] = rowb[i, sl] * af
        pltpu.sync_copy(rowb, out_sh.at[ridx], add=True)
        return 0
    lax.fori_loop(0, nb, body, 0)

    plsc.subcore_barrier()

    @pl.when(s == 0)
    def _():
        pltpu.sync_copy(out_sh, outp_hbm.at[c])


def _combine_body(dp_ref, p_ref, o_ref):
    den = dp_ref[0] + dp_ref[1]
    o_ref[...] = (p_ref[0] + p_ref[1]) / (den + 1e-16)


def _combine(outp, denp):
    blk = 2000
    return pl.pallas_call(
        _combine_body,
        grid=(N // blk,),
        in_specs=[pl.BlockSpec((NC, blk, 1), lambda i: (0, i, 0)),
                  pl.BlockSpec((NC, blk, D), lambda i: (0, i, 0))],
        out_specs=pl.BlockSpec((blk, D), lambda i: (i, 0)),
        out_shape=jax.ShapeDtypeStruct((N, D), jnp.float32),
    )(denp, outp)


def kernel(x, edge_index, W1, a_src1, a_dst1, W2, a_src2, a_dst2):
    src1 = edge_index[0, :NE1]
    dst1 = edge_index[1, :NE1]
    src2 = edge_index[0, NE1:]
    dst2 = edge_index[1, NE1:]

    pad_i = jnp.zeros((P - NE1,), jnp.int32)
    s1 = jnp.concatenate([src1, pad_i])
    d1 = jnp.concatenate([dst1, jnp.full((P - NE1,), N, jnp.int32)])

    xs, xd, ptab = _sc_prep(x, s1, d1)

    Hc2, q_src, q_dst = _dense_stage(xs, xd, d1, W1, a_src1, a_dst1,
                                     W2, a_src2, a_dst2)

    epad = jnp.zeros((EPAD - NE2,), jnp.int32)
    src2p = jnp.concatenate([src2, epad])
    dst2p = jnp.concatenate([dst2, epad])

    denp, outp = _sc_main(src2p, dst2p, ptab, q_src.reshape(P),
                          q_dst.reshape(P), Hc2)
    return _combine(outp, denp[:, :N].reshape(NC, N, 1))


# speedup vs baseline: 108.9975x; 1.0189x over previous
import functools

import jax
import jax.numpy as jnp
from jax import lax
from jax.experimental import pallas as pl
from jax.experimental.pallas import tpu as pltpu
from jax.experimental.pallas import tpu_sc as plsc

N = 10000
D = 128
NE1 = 500
NE2 = 320000 - NE1
P = 512
PAD_SLOT = 511

NC, NS, L = 2, 16, 16
NW = NC * NS
CHUNK = 10112
WDEN = 1264
EPAD = NW * CHUNK
NPT = 10240
NR = 10112
CCAP = CHUNK + L

_mesh = plsc.VectorSubcoreMesh(core_axis_name="c", subcore_axis_name="s")
_NLP = pltpu.CompilerParams(needs_layout_passes=False)


def _wid():
    return lax.axis_index("s") * NC + lax.axis_index("c")


@functools.partial(
    pl.kernel,
    out_type=(
        jax.ShapeDtypeStruct((P, D), jnp.float32),
        jax.ShapeDtypeStruct((P, D), jnp.float32),
        jax.ShapeDtypeStruct((NPT,), jnp.int32),
    ),
    mesh=_mesh,
    compiler_params=_NLP,
    scratch_types=[
        pltpu.VMEM((L,), jnp.int32),
        pltpu.VMEM((L, D), jnp.float32),
        pltpu.VMEM((P,), jnp.int32),
        pltpu.VMEM((NPT,), jnp.int32),
        pltpu.SemaphoreType.DMA,
    ],
)
def _sc_prep(x_hbm, s1_hbm, d1_hbm, xs_hbm, xd_hbm, ptab_hbm,
             idxb, rowb, dstb, ptb, sem):
    wid = _wid()
    base = wid * L
    pltpu.sync_copy(s1_hbm.at[pl.ds(base, L)], idxb)
    pltpu.async_copy(x_hbm.at[idxb], rowb, sem).wait()
    pltpu.sync_copy(rowb, xs_hbm.at[pl.ds(base, L)])
    pltpu.sync_copy(d1_hbm.at[pl.ds(base, L)], idxb)
    pltpu.async_copy(x_hbm.at[idxb], rowb, sem).wait()
    pltpu.sync_copy(rowb, xd_hbm.at[pl.ds(base, L)])

    @pl.when(wid == 0)
    def _():
        pltpu.sync_copy(d1_hbm, dstb)
        fill = jnp.full((L,), PAD_SLOT, jnp.int32)

        def init(i, _):
            ptb[pl.ds(i * L, L)] = fill
            return 0
        lax.fori_loop(0, NPT // L, init, 0)

        lanes = lax.iota(jnp.int32, L)

        def scat(b, _):
            d = dstb[pl.ds(b * L, L)]
            j = jnp.full((L,), b * L, jnp.int32) + lanes
            for l in range(L):
                plsc.store_scatter(ptb, [d], j, mask=lanes == l)
            return 0
        lax.fori_loop(0, P // L, scat, 0)
        pltpu.sync_copy(ptb, ptab_hbm)


def _dense_body(xs_ref, xd_ref, dcol_ref, drow_ref, w1_ref, a1s_ref, a1d_ref,
                w2_ref, a2s_ref, a2d_ref, hc2_ref, qs_ref, qd_ref):
    xs = xs_ref[...]
    xd = xd_ref[...]
    W1 = w1_ref[...]
    W2 = w2_ref[...]
    f32 = jnp.float32
    dot = lambda a, b: jax.lax.dot(a, b, preferred_element_type=f32,
                                   precision=jax.lax.Precision.HIGHEST)
    b1s = dot(W1, a1s_ref[...])
    b1d = dot(W1, a1d_ref[...])
    e1 = dot(xs, b1s) + dot(xd, b1d)
    e1 = jnp.where(e1 >= 0, e1, 0.2 * e1)
    valid = jax.lax.broadcasted_iota(jnp.int32, (P, 1), 0) < NE1
    w1 = jnp.where(valid, jnp.exp(e1), 0.0)
    hs1 = dot(xs, W1)
    M = (dcol_ref[...] == drow_ref[...]).astype(f32)
    dvec = dot(M, w1)
    r = w1 / (dvec + 1e-30)
    Hc = dot(M, r * hs1)
    Hc2 = dot(Hc, W2)
    hc2_ref[...] = Hc2
    qs_ref[...] = dot(Hc2, a2s_ref[...])
    qd_ref[...] = dot(Hc2, a2d_ref[...])


def _dense_stage(xs, xd, d1, W1, a_src1, a_dst1, W2, a_src2, a_dst2):
    f32 = jnp.float32
    out_shapes = (
        jax.ShapeDtypeStruct((P, D), f32),
        jax.ShapeDtypeStruct((P, 1), f32),
        jax.ShapeDtypeStruct((P, 1), f32),
    )
    return pl.pallas_call(_dense_body, out_shape=out_shapes)(
        xs, xd, d1.reshape(P, 1), d1.reshape(1, P),
        W1, a_src1.reshape(D, 1), a_dst1.reshape(D, 1),
        W2, a_src2.reshape(D, 1), a_dst2.reshape(D, 1))


@functools.partial(
    pl.kernel,
    out_type=(
        jax.ShapeDtypeStruct((NC, NPT), jnp.float32),
        jax.ShapeDtypeStruct((NC, NR, D), jnp.float32),
    ),
    mesh=_mesh,
    compiler_params=_NLP,
    scratch_types=[
        pltpu.VMEM((WDEN,), jnp.int32),
        pltpu.VMEM((WDEN,), jnp.int32),
        pltpu.VMEM((NPT,), jnp.int32),
        pltpu.VMEM((P,), jnp.float32),
        pltpu.VMEM((P,), jnp.float32),
        pltpu.VMEM((CCAP,), jnp.int32),
        pltpu.VMEM((CCAP,), jnp.int32),
        pltpu.VMEM((CCAP,), jnp.float32),
        pltpu.VMEM((WDEN,), jnp.int32),
        pltpu.VMEM((WDEN,), jnp.float32),
        pltpu.VMEM((L, D), jnp.float32),
        pltpu.VMEM((L,), jnp.int32),
        pltpu.VMEM((L,), jnp.int32),
        pltpu.SemaphoreType.DMA,
        pltpu.VMEM_SHARED((NPT,), jnp.float32),
        pltpu.VMEM_SHARED((NR, D), jnp.float32),
    ],
)
def _sc_main(src_hbm, dst_hbm, ptab_hbm, qs_hbm, qd_hbm, hc2_hbm,
             denp_hbm, outp_hbm,
             srcb, dstb, ptb, qsb, qdb, cdst, cslot, cxv,
             widx, wval, rowb, ridx, sidx, sem, den_sh, out_sh):
    c = lax.axis_index("c")
    s = lax.axis_index("s")
    wid = s * NC + c
    eb = wid * CHUNK
    lanes = lax.iota(jnp.int32, L)
    zero16 = jnp.zeros((L,), jnp.float32)

    for i in range(L):
        for j in range(D // L):
            rowb[i, pl.ds(j * L, L)] = zero16
    rows_per = NR // NS
    zhs = []
    for k in range(rows_per // L):
        zhs.append(pltpu.async_copy(
            rowb, out_sh.at[pl.ds(s * rows_per + k * L, L)], sem))
    rem = rows_per % L
    zhs.append(pltpu.async_copy(
        rowb.at[pl.ds(0, rem)],
        out_sh.at[pl.ds(s * rows_per + (rows_per // L) * L, rem)], sem))

    @pl.when(s == 0)
    def _():
        def zl(i, _):
            wval[pl.ds(i * L, L)] = zero16
            return 0
        lax.fori_loop(0, 1024 // L, zl, 0)

        def zc(i, _):
            pltpu.sync_copy(wval.at[pl.ds(0, 1024)],
                            den_sh.at[pl.ds(i * 1024, 1024)])
            return 0
        lax.fori_loop(0, NPT // 1024, zc, 0)

    pltpu.sync_copy(ptab_hbm, ptb)
    pltpu.sync_copy(qs_hbm, qsb)
    pltpu.sync_copy(qd_hbm, qdb)
    plsc.subcore_barrier()

    def window(w, cnt_vec):
        pltpu.sync_copy(src_hbm.at[pl.ds(eb + w * WDEN, WDEN)], srcb)
        pltpu.sync_copy(dst_hbm.at[pl.ds(eb + w * WDEN, WDEN)], dstb)

        def batch(k, cv):
            off = k * L
            sv = srcb[pl.ds(off, L)]
            dv = dstb[pl.ds(off, L)]
            ss = plsc.load_gather(ptb, [sv])
            sd = plsc.load_gather(ptb, [dv])
            e = plsc.load_gather(qsb, [ss]) + plsc.load_gather(qdb, [sd])
            e = jnp.where(e >= 0, e, 0.2 * e)
            xv = jnp.exp(e)
            gid = jnp.full((L,), eb + w * WDEN, jnp.int32) + off + lanes
            validm = gid < NE2
            xv = jnp.where(validm, xv, 0.0)
            widx[pl.ds(k * L, L)] = dv
            wval[pl.ds(k * L, L)] = xv
            hot = validm & (ss < NE1)
            pos = cv + plsc.cumsum(hot.astype(jnp.int32)) - 1
            plsc.store_scatter(cdst, [pos], dv, mask=hot)
            plsc.store_scatter(cslot, [pos], ss, mask=hot)
            plsc.store_scatter(cxv, [pos], xv, mask=hot)
            return cv + plsc.all_reduce_population_count(hot)
        cnt_vec = lax.fori_loop(0, WDEN // L, batch, cnt_vec)
        pltpu.sync_copy(wval, den_sh.at[widx], add=True)
        return cnt_vec

    cnt_vec = lax.fori_loop(0, CHUNK // WDEN, window,
                            jnp.zeros((L,), jnp.int32))

    pad_pos = cnt_vec + lanes
    plsc.store_scatter(cdst, [pad_pos], jnp.zeros((L,), jnp.int32))
    plsc.store_scatter(cslot, [pad_pos], jnp.full((L,), PAD_SLOT, jnp.int32))
    plsc.store_scatter(cxv, [pad_pos], zero16)

    for h in zhs:
        h.wait()
    plsc.subcore_barrier()

    dstripe = NPT // NS
    pltpu.sync_copy(den_sh.at[pl.ds(s * dstripe, dstripe)],
                    denp_hbm.at[c, pl.ds(s * dstripe, dstripe)])

    cnt = jnp.max(cnt_vec)
    nb = (cnt + L - 1) // L

    def body(b, _):
        off = b * L
        dv = cdst[pl.ds(off, L)]
        sv = cslot[pl.ds(off, L)]
        xv = cxv[pl.ds(off, L)]
        ridx[...] = dv
        sidx[...] = sv
        pltpu.async_copy(hc2_hbm.at[sidx], rowb, sem).wait()
        for i in range(L):
            af = xv[i]
            for j in range(D // L):
                sl = pl.ds(j * L, L)
                rowb[i, sl] = rowb[i, sl] * af
        pltpu.sync_copy(rowb, out_sh.at[ridx], add=True)
        return 0
    lax.fori_loop(0, nb, body, 0)

    plsc.subcore_barrier()

    pltpu.sync_copy(out_sh.at[pl.ds(s * rows_per, rows_per)],
                    outp_hbm.at[c, pl.ds(s * rows_per, rows_per)])


def _combine_body(dp_ref, p_ref, o_ref):
    den = dp_ref[0] + dp_ref[1]
    o_ref[...] = (p_ref[0] + p_ref[1]) / (den + 1e-16)


def _combine(outp, denp):
    blk = 2000
    return pl.pallas_call(
        _combine_body,
        grid=(N // blk,),
        in_specs=[pl.BlockSpec((NC, blk, 1), lambda i: (0, i, 0)),
                  pl.BlockSpec((NC, blk, D), lambda i: (0, i, 0))],
        out_specs=pl.BlockSpec((blk, D), lambda i: (i, 0)),
        out_shape=jax.ShapeDtypeStruct((N, D), jnp.float32),
    )(denp, outp)


def kernel(x, edge_index, W1, a_src1, a_dst1, W2, a_src2, a_dst2):
    src1 = edge_index[0, :NE1]
    dst1 = edge_index[1, :NE1]
    src2 = edge_index[0, NE1:]
    dst2 = edge_index[1, NE1:]

    pad_i = jnp.zeros((P - NE1,), jnp.int32)
    s1 = jnp.concatenate([src1, pad_i])
    d1 = jnp.concatenate([dst1, jnp.full((P - NE1,), N, jnp.int32)])

    xs, xd, ptab = _sc_prep(x, s1, d1)

    Hc2, q_src, q_dst = _dense_stage(xs, xd, d1, W1, a_src1, a_dst1,
                                     W2, a_src2, a_dst2)

    epad = jnp.zeros((EPAD - NE2,), jnp.int32)
    src2p = jnp.concatenate([src2, epad])
    dst2p = jnp.concatenate([dst2, epad])

    denp, outp = _sc_main(src2p, dst2p, ptab, q_src.reshape(P),
                          q_dst.reshape(P), Hc2)
    return _combine(outp, denp[:, :N].reshape(NC, N, 1))


# speedup vs baseline: 111.3550x; 1.0216x over previous
import functools

import jax
import jax.numpy as jnp
from jax import lax
from jax.experimental import pallas as pl
from jax.experimental.pallas import tpu as pltpu
from jax.experimental.pallas import tpu_sc as plsc

N = 10000
D = 128
NE1 = 500
NE2 = 320000 - NE1
P = 512
PAD_SLOT = 511

NC, NS, L = 2, 16, 16
NW = NC * NS
CHUNK = 10112
WDEN = 1264
EPAD = NW * CHUNK
NPT = 10240
NR = 10112
CCAP = CHUNK + 2 * L

_mesh = plsc.VectorSubcoreMesh(core_axis_name="c", subcore_axis_name="s")
_NLP = pltpu.CompilerParams(needs_layout_passes=False)


def _wid():
    return lax.axis_index("s") * NC + lax.axis_index("c")


@functools.partial(
    pl.kernel,
    out_type=(
        jax.ShapeDtypeStruct((P, D), jnp.float32),
        jax.ShapeDtypeStruct((P, D), jnp.float32),
        jax.ShapeDtypeStruct((NPT,), jnp.int32),
    ),
    mesh=_mesh,
    compiler_params=_NLP,
    scratch_types=[
        pltpu.VMEM((L,), jnp.int32),
        pltpu.VMEM((L, D), jnp.float32),
        pltpu.VMEM((P,), jnp.int32),
        pltpu.VMEM((NPT,), jnp.int32),
        pltpu.SemaphoreType.DMA,
    ],
)
def _sc_prep(x_hbm, s1_hbm, d1_hbm, xs_hbm, xd_hbm, ptab_hbm,
             idxb, rowb, dstb, ptb, sem):
    wid = _wid()
    base = wid * L
    pltpu.sync_copy(s1_hbm.at[pl.ds(base, L)], idxb)
    pltpu.async_copy(x_hbm.at[idxb], rowb, sem).wait()
    pltpu.sync_copy(rowb, xs_hbm.at[pl.ds(base, L)])
    pltpu.sync_copy(d1_hbm.at[pl.ds(base, L)], idxb)
    pltpu.async_copy(x_hbm.at[idxb], rowb, sem).wait()
    pltpu.sync_copy(rowb, xd_hbm.at[pl.ds(base, L)])

    @pl.when(wid == 0)
    def _():
        pltpu.sync_copy(d1_hbm, dstb)
        fill = jnp.full((L,), PAD_SLOT, jnp.int32)

        def init(i, _):
            ptb[pl.ds(i * L, L)] = fill
            return 0
        lax.fori_loop(0, NPT // L, init, 0)

        lanes = lax.iota(jnp.int32, L)

        def scat(b, _):
            d = dstb[pl.ds(b * L, L)]
            j = jnp.full((L,), b * L, jnp.int32) + lanes
            for l in range(L):
                plsc.store_scatter(ptb, [d], j, mask=lanes == l)
            return 0
        lax.fori_loop(0, P // L, scat, 0)
        pltpu.sync_copy(ptb, ptab_hbm)


def _dense_body(xs_ref, xd_ref, dcol_ref, drow_ref, w1_ref, a1s_ref, a1d_ref,
                w2_ref, a2s_ref, a2d_ref, hc2_ref, qs_ref, qd_ref):
    xs = xs_ref[...]
    xd = xd_ref[...]
    W1 = w1_ref[...]
    W2 = w2_ref[...]
    f32 = jnp.float32
    dot = lambda a, b: jax.lax.dot(a, b, preferred_element_type=f32,
                                   precision=jax.lax.Precision.HIGHEST)
    b1s = dot(W1, a1s_ref[...])
    b1d = dot(W1, a1d_ref[...])
    e1 = dot(xs, b1s) + dot(xd, b1d)
    e1 = jnp.where(e1 >= 0, e1, 0.2 * e1)
    valid = jax.lax.broadcasted_iota(jnp.int32, (P, 1), 0) < NE1
    w1 = jnp.where(valid, jnp.exp(e1), 0.0)
    hs1 = dot(xs, W1)
    M = (dcol_ref[...] == drow_ref[...]).astype(f32)
    dvec = dot(M, w1)
    r = w1 / (dvec + 1e-30)
    Hc = dot(M, r * hs1)
    Hc2 = dot(Hc, W2)
    hc2_ref[...] = Hc2
    qs_ref[...] = dot(Hc2, a2s_ref[...])
    qd_ref[...] = dot(Hc2, a2d_ref[...])


def _dense_stage(xs, xd, d1, W1, a_src1, a_dst1, W2, a_src2, a_dst2):
    f32 = jnp.float32
    out_shapes = (
        jax.ShapeDtypeStruct((P, D), f32),
        jax.ShapeDtypeStruct((P, 1), f32),
        jax.ShapeDtypeStruct((P, 1), f32),
    )
    return pl.pallas_call(_dense_body, out_shape=out_shapes)(
        xs, xd, d1.reshape(P, 1), d1.reshape(1, P),
        W1, a_src1.reshape(D, 1), a_dst1.reshape(D, 1),
        W2, a_src2.reshape(D, 1), a_dst2.reshape(D, 1))


@functools.partial(
    pl.kernel,
    out_type=(
        jax.ShapeDtypeStruct((NC, NPT), jnp.float32),
        jax.ShapeDtypeStruct((NC, NR, D), jnp.float32),
    ),
    mesh=_mesh,
    compiler_params=_NLP,
    scratch_types=[
        pltpu.VMEM((WDEN,), jnp.int32),
        pltpu.VMEM((WDEN,), jnp.int32),
        pltpu.VMEM((NPT,), jnp.int32),
        pltpu.VMEM((P,), jnp.float32),
        pltpu.VMEM((P,), jnp.float32),
        pltpu.VMEM((CCAP,), jnp.int32),
        pltpu.VMEM((CCAP,), jnp.float32),
        pltpu.VMEM((WDEN,), jnp.int32),
        pltpu.VMEM((WDEN,), jnp.float32),
        pltpu.VMEM((L, D), jnp.float32),
        pltpu.VMEM((L, D), jnp.float32),
        pltpu.SemaphoreType.DMA,
        pltpu.SemaphoreType.DMA,
        pltpu.VMEM_SHARED((NPT,), jnp.float32),
        pltpu.VMEM_SHARED((NR, D), jnp.float32),
    ],
)
def _sc_main(src_hbm, dst_hbm, ptab_hbm, qs_hbm, qd_hbm, hc2_hbm,
             denp_hbm, outp_hbm,
             srcb, dstb, ptb, qsb, qdb, cpak, cxv,
             widx, wval, rowb, rowb2, sem, sem2, den_sh, out_sh):
    c = lax.axis_index("c")
    s = lax.axis_index("s")
    wid = s * NC + c
    eb = wid * CHUNK
    lanes = lax.iota(jnp.int32, L)
    zero16 = jnp.zeros((L,), jnp.float32)

    for i in range(L):
        for j in range(D // L):
            rowb[i, pl.ds(j * L, L)] = zero16
    rows_per = NR // NS
    zhs = []
    for k in range(rows_per // L):
        zhs.append(pltpu.async_copy(
            rowb, out_sh.at[pl.ds(s * rows_per + k * L, L)], sem))
    rem = rows_per % L
    zhs.append(pltpu.async_copy(
        rowb.at[pl.ds(0, rem)],
        out_sh.at[pl.ds(s * rows_per + (rows_per // L) * L, rem)], sem))

    @pl.when(s == 0)
    def _():
        def zl(i, _):
            wval[pl.ds(i * L, L)] = zero16
            return 0
        lax.fori_loop(0, 1024 // L, zl, 0)

        def zc(i, _):
            pltpu.sync_copy(wval.at[pl.ds(0, 1024)],
                            den_sh.at[pl.ds(i * 1024, 1024)])
            return 0
        lax.fori_loop(0, NPT // 1024, zc, 0)

    pltpu.sync_copy(ptab_hbm, ptb)
    pltpu.sync_copy(qs_hbm, qsb)
    pltpu.sync_copy(qd_hbm, qdb)
    plsc.subcore_barrier()

    def window(w, cnt_vec):
        pltpu.sync_copy(src_hbm.at[pl.ds(eb + w * WDEN, WDEN)], srcb)
        pltpu.sync_copy(dst_hbm.at[pl.ds(eb + w * WDEN, WDEN)], dstb)

        def batch(k, cv):
            off = k * L
            sv = srcb[pl.ds(off, L)]
            dv = dstb[pl.ds(off, L)]
            ss = plsc.load_gather(ptb, [sv])
            sd = plsc.load_gather(ptb, [dv])
            e = plsc.load_gather(qsb, [ss]) + plsc.load_gather(qdb, [sd])
            e = jnp.where(e >= 0, e, 0.2 * e)
            xv = jnp.exp(e)
            gid = jnp.full((L,), eb + w * WDEN, jnp.int32) + off + lanes
            validm = gid < NE2
            xv = jnp.where(validm, xv, 0.0)
            widx[pl.ds(k * L, L)] = dv
            wval[pl.ds(k * L, L)] = xv
            hot = validm & (ss < NE1)
            pos = cv + plsc.cumsum(hot.astype(jnp.int32)) - 1
            plsc.store_scatter(cpak, [pos], dv * 512 + ss, mask=hot)
            plsc.store_scatter(cxv, [pos], xv, mask=hot)
            return cv + plsc.all_reduce_population_count(hot)
        cnt_vec = lax.fori_loop(0, WDEN // L, batch, cnt_vec)
        pltpu.sync_copy(wval, den_sh.at[widx], add=True)
        return cnt_vec

    cnt_vec = lax.fori_loop(0, CHUNK // WDEN, window,
                            jnp.zeros((L,), jnp.int32))

    for pp in range(2):
        pad_pos = cnt_vec + jnp.full((L,), pp * L, jnp.int32) + lanes
        plsc.store_scatter(cpak, [pad_pos],
                           jnp.full((L,), PAD_SLOT, jnp.int32))
        plsc.store_scatter(cxv, [pad_pos], zero16)

    for h in zhs:
        h.wait()
    plsc.subcore_barrier()

    dstripe = NPT // NS
    pltpu.sync_copy(den_sh.at[pl.ds(s * dstripe, dstripe)],
                    denp_hbm.at[c, pl.ds(s * dstripe, dstripe)])

    cnt = jnp.max(cnt_vec)
    npair = (cnt + 2 * L - 1) // (2 * L)

    def body(b, _):
        off = 2 * b * L
        pk = cpak[pl.ds(off, L)]
        dv = jnp.right_shift(pk, 9)
        sv = jnp.bitwise_and(pk, 511)
        xv = cxv[pl.ds(off, L)]
        h0 = pltpu.async_copy(hc2_hbm.at[sv], rowb, sem)
        pk2 = cpak[pl.ds(off + L, L)]
        dv2 = jnp.right_shift(pk2, 9)
        sv2 = jnp.bitwise_and(pk2, 511)
        xv2 = cxv[pl.ds(off + L, L)]
        h1 = pltpu.async_copy(hc2_hbm.at[sv2], rowb2, sem2)
        h0.wait()
        for i in range(L):
            af = xv[i]
            for j in range(D // L):
                sl = pl.ds(j * L, L)
                rowb[i, sl] = rowb[i, sl] * af
        pltpu.sync_copy(rowb, out_sh.at[dv], add=True)
        h1.wait()
        for i in range(L):
            af = xv2[i]
            for j in range(D // L):
                sl = pl.ds(j * L, L)
                rowb2[i, sl] = rowb2[i, sl] * af
        pltpu.sync_copy(rowb2, out_sh.at[dv2], add=True)
        return 0
    lax.fori_loop(0, npair, body, 0)

    plsc.subcore_barrier()

    pltpu.sync_copy(out_sh.at[pl.ds(s * rows_per, rows_per)],
                    outp_hbm.at[c, pl.ds(s * rows_per, rows_per)])


def _combine_body(dp_ref, p_ref, o_ref):
    den = dp_ref[0] + dp_ref[1]
    o_ref[...] = (p_ref[0] + p_ref[1]) / (den + 1e-16)


def _combine(outp, denp):
    blk = 2000
    return pl.pallas_call(
        _combine_body,
        grid=(N // blk,),
        in_specs=[pl.BlockSpec((NC, blk, 1), lambda i: (0, i, 0)),
                  pl.BlockSpec((NC, blk, D), lambda i: (0, i, 0))],
        out_specs=pl.BlockSpec((blk, D), lambda i: (i, 0)),
        out_shape=jax.ShapeDtypeStruct((N, D), jnp.float32),
    )(denp, outp)


def kernel(x, edge_index, W1, a_src1, a_dst1, W2, a_src2, a_dst2):
    src1 = edge_index[0, :NE1]
    dst1 = edge_index[1, :NE1]
    src2 = edge_index[0, NE1:]
    dst2 = edge_index[1, NE1:]

    pad_i = jnp.zeros((P - NE1,), jnp.int32)
    s1 = jnp.concatenate([src1, pad_i])
    d1 = jnp.concatenate([dst1, jnp.full((P - NE1,), N, jnp.int32)])

    xs, xd, ptab = _sc_prep(x, s1, d1)

    Hc2, q_src, q_dst = _dense_stage(xs, xd, d1, W1, a_src1, a_dst1,
                                     W2, a_src2, a_dst2)

    epad = jnp.zeros((EPAD - NE2,), jnp.int32)
    src2p = jnp.concatenate([src2, epad])
    dst2p = jnp.concatenate([dst2, epad])

    denp, outp = _sc_main(src2p, dst2p, ptab, q_src.reshape(P),
                          q_dst.reshape(P), Hc2)
    return _combine(outp, denp[:, :N].reshape(NC, N, 1))
